# baseline (speedup 1.0000x reference)
import functools

import jax
import jax.numpy as jnp
from jax import lax
from jax.experimental import pallas as pl
from jax.experimental.pallas import tpu as pltpu
from jax.experimental.pallas import tpu_sc as plsc

_N = 10000
_E = 160000
_D = 128
_H = 8
_DK = 16
_ACC_W = 144
_CHUNK = 80
_NCHUNK = 125
_BLK = 2000
_NBLK = (2 * _N) // _BLK

_f32 = jnp.float32
_i32 = jnp.int32


def _proj_body(h_ref, kw_ref, kb_ref, vw_ref, vb_ref, qw_ref, qb_ref,
               att_ref, msg_ref, pri_ref, ko_ref, vo_ref, qo_ref):
    h = h_ref[...]
    k = jnp.dot(h, kw_ref[0], preferred_element_type=_f32) + kb_ref[0]
    v = jnp.dot(h, vw_ref[0], preferred_element_type=_f32) + vb_ref[0]
    q = jnp.dot(h, qw_ref[0], preferred_element_type=_f32) + qb_ref[0]
    att = att_ref[0]
    msg = msg_ref[0]
    kparts = []
    vparts = []
    for hh in range(_H):
        sl = slice(hh * _DK, (hh + 1) * _DK)
        kparts.append(jnp.dot(k[:, sl], att[hh], preferred_element_type=_f32))
        vparts.append(jnp.dot(v[:, sl], msg[hh], preferred_element_type=_f32))
    ko_ref[...] = jnp.concatenate(kparts, axis=1)
    vo_ref[...] = jnp.concatenate(vparts, axis=1)
    scale = jnp.repeat(pri_ref[0], _DK, axis=1) * 0.25
    qo_ref[...] = q * scale


def _run_proj(h_cat, kw, kb, vw, vb, qw, qb, att, msg, pri):
    npb = _NBLK // 2
    row_spec = pl.BlockSpec((_BLK, _D), lambda i: (i, 0))
    w_spec = pl.BlockSpec((1, _D, _D), lambda i: (i // npb, 0, 0))
    b_spec = pl.BlockSpec((1, 1, _D), lambda i: (i // npb, 0, 0))
    hm_spec = pl.BlockSpec((1, _H, _DK, _DK), lambda i: (i // npb, 0, 0, 0))
    pri_spec = pl.BlockSpec((1, 1, _H), lambda i: (i // npb, 0, 0))
    out = jax.ShapeDtypeStruct((2 * _N, _D), _f32)
    return pl.pallas_call(
        _proj_body,
        grid=(_NBLK,),
        in_specs=[row_spec, w_spec, b_spec, w_spec, b_spec, w_spec, b_spec,
                  hm_spec, hm_spec, pri_spec],
        out_specs=[row_spec, row_spec, row_spec],
        out_shape=[out, out, out],
    )(h_cat, kw, kb, vw, vb, qw, qb, att, msg, pri)


_NEX = _N // 16


_C = 32
_NCH = 312


def _edge_body(kcat, qcat, vcat, srcg, dstq, dstl, out_msg, out_ex,
               accm, acce, sidx, qidx, didx, d8,
               kbuf, qbuf, vbuf, tsidx, tqidx, tdidx, td8, zbuf,
               ksem, qsem, vsem, msem, esem):
    cid = lax.axis_index("c")
    sid = lax.axis_index("s")
    iota16 = lax.iota(_i32, 16)
    zero16 = jnp.zeros((16,), _f32)

    for r in range(16):
        for c in range(_D // 16):
            zbuf[r, pl.ds(16 * c, 16)] = zero16

    rbase = sid * 624

    def zacc(i, carry):
        pltpu.sync_copy(zbuf, accm.at[pl.ds(rbase + i * 16, 16)])
        return carry

    lax.fori_loop(0, 39, zacc, 0)

    @pl.when(sid == 15)
    def _():
        pltpu.sync_copy(zbuf, accm.at[pl.ds(9984, 16)])

    def zex(i, carry):
        pltpu.sync_copy(zbuf, acce.at[pl.ds(sid * 80 + i * 16, 16)])
        return carry

    lax.fori_loop(0, 5, zex, 0)

    plsc.subcore_barrier()

    ebase = cid * _E + sid * 10000

    def fetch_idx(jj, bb):
        off = ebase + jj * _C
        pltpu.sync_copy(srcg.at[pl.ds(off, _C)], sidx.at[bb])
        pltpu.sync_copy(dstq.at[pl.ds(off, _C)], qidx.at[bb])
        pltpu.sync_copy(dstl.at[pl.ds(off, _C)], didx.at[bb])
        for g in range(_C // 16):
            dv = didx[bb, pl.ds(16 * g, 16)]
            d8[bb, pl.ds(16 * g, 16)] = lax.shift_right_logical(dv, 3)

    def start_gathers(bb):
        pltpu.async_copy(kcat.at[sidx.at[bb]], kbuf.at[bb], ksem.at[bb])
        pltpu.async_copy(qcat.at[qidx.at[bb]], qbuf.at[bb], qsem.at[bb])
        pltpu.async_copy(vcat.at[sidx.at[bb]], vbuf.at[bb], vsem.at[bb])

    def drain(sem, bb):
        pltpu.make_async_copy(kcat.at[pl.ds(0, _C)], kbuf.at[bb],
                              sem.at[bb]).wait()

    def compute_chunk(kb, qb, vb, db):
        def group(g, gcarry):
            dvvec = db[pl.ds(16 * g, 16)]
            for r in range(16):
                i = 16 * g + r
                exacc = jnp.zeros((16,), _f32)
                for hh in range(_H):
                    k = kb[i, pl.ds(_DK * hh, _DK)]
                    q = qb[i, pl.ds(_DK * hh, _DK)]
                    t = k * q
                    for sh in (8, 4, 2, 1):
                        t = t + t.at[jnp.bitwise_xor(iota16, sh)].get(
                            mode='promise_in_bounds')
                    ex = jnp.exp(t)
                    v = vb[i, pl.ds(_DK * hh, _DK)]
                    kb[i, pl.ds(_DK * hh, _DK)] = ex * v
                    exacc = jnp.where(iota16 == hh, ex, exacc)
                for c in range(_D // 16):
                    qb[i, pl.ds(16 * c, 16)] = zero16
                dcol = lax.bitwise_and(dvvec[r], 7) * 16
                qb[i, pl.ds(dcol, 16)] = exacc
            return gcarry

        lax.fori_loop(0, _C // 16, group, 0)

    fetch_idx(0, 0)
    start_gathers(0)

    def chunk(j, carry):
        p = lax.bitwise_and(j, 1)
        np_ = 1 - p

        @pl.when(j >= 1)
        def _():
            drain(msem, np_)
            drain(esem, np_)

        @pl.when(j + 1 <= _NCH - 1)
        def _():
            fetch_idx(j + 1, np_)
            start_gathers(np_)

        drain(ksem, p)
        drain(qsem, p)
        drain(vsem, p)
        compute_chunk(kbuf.at[p], qbuf.at[p], vbuf.at[p], didx.at[p])
        pltpu.async_copy(kbuf.at[p], accm.at[didx.at[p]], msem.at[p],
                         add=True)
        pltpu.async_copy(qbuf.at[p], acce.at[d8.at[p]], esem.at[p],
                         add=True)
        return carry

    lax.fori_loop(0, _NCH, chunk, 0)

    lastp = (_NCH - 1) & 1
    drain(msem, lastp)
    drain(esem, lastp)

    toff = ebase + _NCH * _C
    pltpu.sync_copy(srcg.at[pl.ds(toff, 16)], tsidx)
    pltpu.sync_copy(dstq.at[pl.ds(toff, 16)], tqidx)
    pltpu.sync_copy(dstl.at[pl.ds(toff, 16)], tdidx)
    dv = tdidx[pl.ds(0, 16)]
    td8[pl.ds(0, 16)] = lax.shift_right_logical(dv, 3)
    ck = pltpu.async_copy(kcat.at[tsidx], kbuf.at[0, pl.ds(0, 16)],
                          ksem.at[0])
    cq = pltpu.async_copy(qcat.at[tqidx], qbuf.at[0, pl.ds(0, 16)],
                          qsem.at[0])
    cv = pltpu.async_copy(vcat.at[tsidx], vbuf.at[0, pl.ds(0, 16)],
                          vsem.at[0])
    ck.wait()
    cq.wait()
    cv.wait()
    dvvec = tdidx[pl.ds(0, 16)]
    for i in range(16):
        exacc = jnp.zeros((16,), _f32)
        for hh in range(_H):
            k = kbuf[0, i, pl.ds(_DK * hh, _DK)]
            q = qbuf[0, i, pl.ds(_DK * hh, _DK)]
            t = k * q
            for sh in (8, 4, 2, 1):
                t = t + t.at[jnp.bitwise_xor(iota16, sh)].get(
                    mode='promise_in_bounds')
            ex = jnp.exp(t)
            v = vbuf[0, i, pl.ds(_DK * hh, _DK)]
            kbuf[0, i, pl.ds(_DK * hh, _DK)] = ex * v
            exacc = jnp.where(iota16 == hh, ex, exacc)
        for c in range(_D // 16):
            qbuf[0, i, pl.ds(16 * c, 16)] = zero16
        dcol = lax.bitwise_and(dvvec[i], 7) * 16
        qbuf[0, i, pl.ds(dcol, 16)] = exacc
    pltpu.sync_copy(kbuf.at[0, pl.ds(0, 16)], accm.at[tdidx], add=True)
    pltpu.sync_copy(qbuf.at[0, pl.ds(0, 16)], acce.at[td8], add=True)

    plsc.subcore_barrier()

    pltpu.sync_copy(accm.at[pl.ds(rbase, 624)],
                    out_msg.at[pl.ds(cid * _N + rbase, 624)])

    @pl.when(sid == 15)
    def _():
        pltpu.sync_copy(accm.at[pl.ds(9984, 16)],
                        out_msg.at[pl.ds(cid * _N + 9984, 16)])

    pltpu.sync_copy(acce.at[pl.ds(sid * 80, 80)],
                    out_ex.at[pl.ds(cid * 1280 + sid * 80, 80)])


def _run_edges(kcat, qcat, vcat, srcg, dstq, dstl):
    mesh = plsc.VectorSubcoreMesh(core_axis_name="c", subcore_axis_name="s")
    fn = pl.kernel(
        _edge_body,
        out_type=[jax.ShapeDtypeStruct((2 * _N, _D), _f32),
                  jax.ShapeDtypeStruct((2 * 1280, _D), _f32)],
        mesh=mesh,
        scratch_types=[
            pltpu.VMEM_SHARED((_N, _D), _f32),
            pltpu.VMEM_SHARED((1280, _D), _f32),
            pltpu.VMEM((2, _C), _i32),
            pltpu.VMEM((2, _C), _i32),
            pltpu.VMEM((2, _C), _i32),
            pltpu.VMEM((2, _C), _i32),
            pltpu.VMEM((2, _C, _D), _f32),
            pltpu.VMEM((2, _C, _D), _f32),
            pltpu.VMEM((2, _C, _D), _f32),
            pltpu.VMEM((16,), _i32),
            pltpu.VMEM((16,), _i32),
            pltpu.VMEM((16,), _i32),
            pltpu.VMEM((16,), _i32),
            pltpu.VMEM((16, _D), _f32),
            pltpu.SemaphoreType.DMA((2,)),
            pltpu.SemaphoreType.DMA((2,)),
            pltpu.SemaphoreType.DMA((2,)),
            pltpu.SemaphoreType.DMA((2,)),
            pltpu.SemaphoreType.DMA((2,)),
        ],
    )
    return fn(kcat, qcat, vcat, srcg, dstq, dstl)


def _update_body(eo_ref, s_ref, h_ref, aw_ref, ab_ref, sk_ref, ls_ref,
                 lb_ref, out_ref):
    agg = eo_ref[...]
    s = s_ref[...]
    den = jnp.repeat(s, _DK, axis=1) + 1e-9
    trans = jnp.dot(agg / den, aw_ref[0], preferred_element_type=_f32) \
        + ab_ref[0]
    alpha = jax.nn.sigmoid(sk_ref[0, 0, 0])
    out = trans * alpha + h_ref[...] * (1.0 - alpha)
    mu = jnp.mean(out, axis=1, keepdims=True)
    var = jnp.mean((out - mu) ** 2, axis=1, keepdims=True)
    out_ref[...] = (out - mu) * lax.rsqrt(var + 1e-5) * ls_ref[0] + lb_ref[0]


def _run_update(edge_msg, s_cat, h_cat, aw, ab, sk, ls, lb):
    npb = _NBLK // 2
    row_spec = pl.BlockSpec((_BLK, _D), lambda i: (i, 0))
    s_spec = pl.BlockSpec((_BLK, _H), lambda i: (i, 0))
    w_spec = pl.BlockSpec((1, _D, _D), lambda i: (i // npb, 0, 0))
    b_spec = pl.BlockSpec((1, 1, _D), lambda i: (i // npb, 0, 0))
    sk_spec = pl.BlockSpec((1, 1, 1), lambda i: (i // npb, 0, 0))
    return pl.pallas_call(
        _update_body,
        grid=(_NBLK,),
        in_specs=[row_spec, s_spec, row_spec, w_spec, b_spec, sk_spec,
                  b_spec, b_spec],
        out_specs=row_spec,
        out_shape=jax.ShapeDtypeStruct((2 * _N, _D), _f32),
    )(edge_msg, s_cat, h_cat, aw, ab, sk, ls, lb)


def kernel(h_user, h_item, params, edge_clicks, edge_clicked_by):
    p = params
    h_cat = jnp.concatenate([h_user, h_item], axis=0)

    kw = jnp.stack([p['k_w_user'], p['k_w_item']])
    kb = jnp.stack([p['k_b_user'], p['k_b_item']])[:, None, :]
    vw = jnp.stack([p['v_w_user'], p['v_w_item']])
    vb = jnp.stack([p['v_b_user'], p['v_b_item']])[:, None, :]
    qw = jnp.stack([p['q_w_user'], p['q_w_item']])
    qb = jnp.stack([p['q_b_user'], p['q_b_item']])[:, None, :]
    att = jnp.stack([p['att_clicks'], p['att_clicked_by']])
    msg = jnp.stack([p['msg_clicks'], p['msg_clicked_by']])
    pri = jnp.stack([p['pri_clicked_by'], p['pri_clicks']])[:, None, :]

    kcat, vcat, qcat = _run_proj(h_cat, kw, kb, vw, vb, qw, qb, att, msg, pri)

    src_c = edge_clicks[0].astype(_i32)
    dst_c = edge_clicks[1].astype(_i32)
    src_cb = edge_clicked_by[0].astype(_i32)
    dst_cb = edge_clicked_by[1].astype(_i32)
    srcg = jnp.concatenate([src_cb + _N, src_c])
    dstq = jnp.concatenate([dst_cb, dst_c + _N])
    dstl = jnp.concatenate([dst_cb, dst_c])

    edge_msg, edge_ex = _run_edges(kcat, qcat, vcat, srcg, dstq, dstl)
    nrows = _N // 8
    epk = jnp.concatenate([edge_ex[:nrows], edge_ex[1280:1280 + nrows]])
    s_cat = epk.reshape(2 * nrows, 8, 16)[:, :, :_H].reshape(2 * _N, _H)

    aw = jnp.stack([p['a_w_user'], p['a_w_item']])
    ab = jnp.stack([p['a_b_user'], p['a_b_item']])[:, None, :]
    sk = jnp.stack([p['skip_user'], p['skip_item']])[:, None, :]
    ls = jnp.stack([p['ln_s_user'], p['ln_s_item']])[:, None, :]
    lb = jnp.stack([p['ln_b_user'], p['ln_b_item']])[:, None, :]

    new_cat = _run_update(edge_msg, s_cat, h_cat, aw, ab, sk, ls, lb)
    return new_cat[:_N], new_cat[_N:]


# speedup vs baseline: 3.4453x; 3.4453x over previous
import functools

import jax
import jax.numpy as jnp
from jax import lax
from jax.experimental import pallas as pl
from jax.experimental.pallas import tpu as pltpu
from jax.experimental.pallas import tpu_sc as plsc

_N = 10000
_E = 160000
_D = 128
_H = 8
_DK = 16
_ACC_W = 144
_CHUNK = 80
_NCHUNK = 125
_BLK = 2000
_NBLK = (2 * _N) // _BLK

_f32 = jnp.float32
_i32 = jnp.int32


def _proj_body(h_ref, kw_ref, kb_ref, vw_ref, vb_ref, qw_ref, qb_ref,
               att_ref, msg_ref, pri_ref, ko_ref, vo_ref, qo_ref):
    h = h_ref[...]
    k = jnp.dot(h, kw_ref[0], preferred_element_type=_f32) + kb_ref[0]
    v = jnp.dot(h, vw_ref[0], preferred_element_type=_f32) + vb_ref[0]
    q = jnp.dot(h, qw_ref[0], preferred_element_type=_f32) + qb_ref[0]
    att = att_ref[0]
    msg = msg_ref[0]
    kparts = []
    vparts = []
    for hh in range(_H):
        sl = slice(hh * _DK, (hh + 1) * _DK)
        kparts.append(jnp.dot(k[:, sl], att[hh], preferred_element_type=_f32))
        vparts.append(jnp.dot(v[:, sl], msg[hh], preferred_element_type=_f32))
    ko_ref[...] = jnp.concatenate(kparts, axis=1)
    vo_ref[...] = jnp.concatenate(vparts, axis=1)
    scale = jnp.repeat(pri_ref[0], _DK, axis=1) * 0.25
    qo_ref[...] = q * scale


def _run_proj(h_cat, kw, kb, vw, vb, qw, qb, att, msg, pri):
    npb = _NBLK // 2
    row_spec = pl.BlockSpec((_BLK, _D), lambda i: (i, 0))
    w_spec = pl.BlockSpec((1, _D, _D), lambda i: (i // npb, 0, 0))
    b_spec = pl.BlockSpec((1, 1, _D), lambda i: (i // npb, 0, 0))
    hm_spec = pl.BlockSpec((1, _H, _DK, _DK), lambda i: (i // npb, 0, 0, 0))
    pri_spec = pl.BlockSpec((1, 1, _H), lambda i: (i // npb, 0, 0))
    out = jax.ShapeDtypeStruct((2 * _N, _D), _f32)
    return pl.pallas_call(
        _proj_body,
        grid=(_NBLK,),
        in_specs=[row_spec, w_spec, b_spec, w_spec, b_spec, w_spec, b_spec,
                  hm_spec, hm_spec, pri_spec],
        out_specs=[row_spec, row_spec, row_spec],
        out_shape=[out, out, out],
    )(h_cat, kw, kb, vw, vb, qw, qb, att, msg, pri)


_NEX = _N // 16


def _edge_body(kcat, qcat, vcat, idx_all, out_msg, out_ex,
               accm, acce, ixbuf, didx, kbuf, qbuf, vbuf,
               d8, zbuf, ksem, qsem, vsem):
    cid = lax.axis_index("c")
    sid = lax.axis_index("s")
    iota16 = lax.iota(_i32, 16)
    zero16 = jnp.zeros((16,), _f32)

    for r in range(16):
        for c in range(_D // 16):
            zbuf[r, pl.ds(16 * c, 16)] = zero16

    rbase = sid * 624

    def zacc(i, carry):
        pltpu.sync_copy(zbuf, accm.at[pl.ds(rbase + i * 16, 16)])
        return carry

    lax.fori_loop(0, 39, zacc, 0)

    @pl.when(sid == 15)
    def _():
        pltpu.sync_copy(zbuf, accm.at[pl.ds(9984, 16)])

    def zex(i, carry):
        pltpu.sync_copy(zbuf, acce.at[pl.ds(sid * 80 + i * 16, 16)])
        return carry

    lax.fori_loop(0, 5, zex, 0)

    plsc.subcore_barrier()

    ibase = (cid * 16 + sid) * _NCHUNK * (3 * _CHUNK)

    def chunk(j, carry):
        pltpu.sync_copy(idx_all.at[pl.ds(ibase + j * (3 * _CHUNK),
                                         3 * _CHUNK)], ixbuf)
        ck = pltpu.async_copy(kcat.at[ixbuf.at[pl.ds(0, _CHUNK)]],
                              kbuf, ksem)
        cq = pltpu.async_copy(qcat.at[ixbuf.at[pl.ds(_CHUNK, _CHUNK)]],
                              qbuf, qsem)
        cv = pltpu.async_copy(vcat.at[ixbuf.at[pl.ds(0, _CHUNK)]],
                              vbuf, vsem)
        for g in range(_CHUNK // 16):
            dv = ixbuf[pl.ds(2 * _CHUNK + 16 * g, 16)]
            didx[pl.ds(16 * g, 16)] = dv
            d8[pl.ds(16 * g, 16)] = lax.shift_right_logical(dv, 3)
        ck.wait()
        cq.wait()
        cv.wait()

        def bfly(x, sh):
            return x + x.at[jnp.bitwise_xor(iota16, sh)].get(
                mode='promise_in_bounds')

        m8 = iota16 < 8
        m4 = lax.bitwise_and(iota16, 7) < 4
        m2 = lax.bitwise_and(iota16, 3) < 2
        _LA = (0, 8, 4, 12, 2, 10, 6, 14)
        lanev = (lax.bitwise_and(iota16, 1) * 8
                 + lax.bitwise_and(iota16, 2) * 2
                 + lax.shift_right_logical(lax.bitwise_and(iota16, 4), 1))

        def group(g, gcarry):
            dvvec = didx[pl.ds(16 * g, 16)]
            for r in range(16):
                i = 16 * g + r
                pr = []
                for hh in range(_H):
                    k = kbuf[i, pl.ds(_DK * hh, _DK)]
                    q = qbuf[i, pl.ds(_DK * hh, _DK)]
                    pr.append(k * q)
                a = [bfly(x, 8) for x in pr]
                b = [jnp.where(m8, a[0], a[1]), jnp.where(m8, a[2], a[3]),
                     jnp.where(m8, a[4], a[5]), jnp.where(m8, a[6], a[7])]
                c = [bfly(x, 4) for x in b]
                d = [jnp.where(m4, c[0], c[1]), jnp.where(m4, c[2], c[3])]
                e = [bfly(x, 2) for x in d]
                f = jnp.where(m2, e[0], e[1])
                exall = jnp.exp(bfly(f, 1))
                for hh in range(_H):
                    exh = exall.at[jnp.full((16,), _LA[hh], _i32)].get(
                        mode='promise_in_bounds')
                    v = vbuf[i, pl.ds(_DK * hh, _DK)]
                    kbuf[i, pl.ds(_DK * hh, _DK)] = exh * v
                exacc = jnp.where(
                    m8,
                    exall.at[lanev].get(mode='promise_in_bounds'),
                    zero16)
                for cc in range(_D // 16):
                    qbuf[i, pl.ds(16 * cc, 16)] = zero16
                dcol = lax.bitwise_and(dvvec[r], 7) * 16
                qbuf[i, pl.ds(dcol, 16)] = exacc
            return gcarry

        lax.fori_loop(0, _CHUNK // 16, group, 0)
        pltpu.sync_copy(kbuf, accm.at[didx], add=True)
        pltpu.sync_copy(qbuf, acce.at[d8], add=True)
        return carry

    lax.fori_loop(0, _NCHUNK, chunk, 0)
    plsc.subcore_barrier()

    pltpu.sync_copy(accm.at[pl.ds(rbase, 624)],
                    out_msg.at[pl.ds(cid * _N + rbase, 624)])

    @pl.when(sid == 15)
    def _():
        pltpu.sync_copy(accm.at[pl.ds(9984, 16)],
                        out_msg.at[pl.ds(cid * _N + 9984, 16)])

    pltpu.sync_copy(acce.at[pl.ds(sid * 80, 80)],
                    out_ex.at[pl.ds(cid * 1280 + sid * 80, 80)])


def _run_edges(kcat, qcat, vcat, idx_all):
    mesh = plsc.VectorSubcoreMesh(core_axis_name="c", subcore_axis_name="s")
    fn = pl.kernel(
        _edge_body,
        out_type=[jax.ShapeDtypeStruct((2 * _N, _D), _f32),
                  jax.ShapeDtypeStruct((2 * 1280, _D), _f32)],
        mesh=mesh,
        scratch_types=[
            pltpu.VMEM_SHARED((_N, _D), _f32),
            pltpu.VMEM_SHARED((1280, _D), _f32),
            pltpu.VMEM((3 * _CHUNK,), _i32),
            pltpu.VMEM((_CHUNK,), _i32),
            pltpu.VMEM((_CHUNK, _D), _f32),
            pltpu.VMEM((_CHUNK, _D), _f32),
            pltpu.VMEM((_CHUNK, _D), _f32),
            pltpu.VMEM((_CHUNK,), _i32),
            pltpu.VMEM((16, _D), _f32),
            pltpu.SemaphoreType.DMA,
            pltpu.SemaphoreType.DMA,
            pltpu.SemaphoreType.DMA,
        ],
    )
    return fn(kcat, qcat, vcat, idx_all)


def _update_body(eo_ref, s_ref, h_ref, aw_ref, ab_ref, sk_ref, ls_ref,
                 lb_ref, out_ref):
    agg = eo_ref[...]
    s = s_ref[...]
    den = jnp.repeat(s, _DK, axis=1) + 1e-9
    trans = jnp.dot(agg / den, aw_ref[0], preferred_element_type=_f32) \
        + ab_ref[0]
    alpha = jax.nn.sigmoid(sk_ref[0, 0, 0])
    out = trans * alpha + h_ref[...] * (1.0 - alpha)
    mu = jnp.mean(out, axis=1, keepdims=True)
    var = jnp.mean((out - mu) ** 2, axis=1, keepdims=True)
    out_ref[...] = (out - mu) * lax.rsqrt(var + 1e-5) * ls_ref[0] + lb_ref[0]


def _run_update(edge_msg, s_cat, h_cat, aw, ab, sk, ls, lb):
    npb = _NBLK // 2
    row_spec = pl.BlockSpec((_BLK, _D), lambda i: (i, 0))
    s_spec = pl.BlockSpec((_BLK, _H), lambda i: (i, 0))
    w_spec = pl.BlockSpec((1, _D, _D), lambda i: (i // npb, 0, 0))
    b_spec = pl.BlockSpec((1, 1, _D), lambda i: (i // npb, 0, 0))
    sk_spec = pl.BlockSpec((1, 1, 1), lambda i: (i // npb, 0, 0))
    return pl.pallas_call(
        _update_body,
        grid=(_NBLK,),
        in_specs=[row_spec, s_spec, row_spec, w_spec, b_spec, sk_spec,
                  b_spec, b_spec],
        out_specs=row_spec,
        out_shape=jax.ShapeDtypeStruct((2 * _N, _D), _f32),
    )(edge_msg, s_cat, h_cat, aw, ab, sk, ls, lb)


def kernel(h_user, h_item, params, edge_clicks, edge_clicked_by):
    p = params
    h_cat = jnp.concatenate([h_user, h_item], axis=0)

    kw = jnp.stack([p['k_w_user'], p['k_w_item']])
    kb = jnp.stack([p['k_b_user'], p['k_b_item']])[:, None, :]
    vw = jnp.stack([p['v_w_user'], p['v_w_item']])
    vb = jnp.stack([p['v_b_user'], p['v_b_item']])[:, None, :]
    qw = jnp.stack([p['q_w_user'], p['q_w_item']])
    qb = jnp.stack([p['q_b_user'], p['q_b_item']])[:, None, :]
    att = jnp.stack([p['att_clicks'], p['att_clicked_by']])
    msg = jnp.stack([p['msg_clicks'], p['msg_clicked_by']])
    pri = jnp.stack([p['pri_clicked_by'], p['pri_clicks']])[:, None, :]

    kcat, vcat, qcat = _run_proj(h_cat, kw, kb, vw, vb, qw, qb, att, msg, pri)

    src_c = edge_clicks[0].astype(_i32)
    dst_c = edge_clicks[1].astype(_i32)
    src_cb = edge_clicked_by[0].astype(_i32)
    dst_cb = edge_clicked_by[1].astype(_i32)
    srcg = jnp.concatenate([src_cb + _N, src_c])
    dstq = jnp.concatenate([dst_cb, dst_c + _N])
    dstl = jnp.concatenate([dst_cb, dst_c])
    shp = (2 * 16 * _NCHUNK, _CHUNK)
    idx_all = jnp.stack([srcg.reshape(shp), dstq.reshape(shp),
                         dstl.reshape(shp)], axis=1).reshape(-1)

    edge_msg, edge_ex = _run_edges(kcat, qcat, vcat, idx_all)
    nrows = _N // 8
    epk = jnp.concatenate([edge_ex[:nrows], edge_ex[1280:1280 + nrows]])
    s_cat = epk.reshape(2 * nrows, 8, 16)[:, :, :_H].reshape(2 * _N, _H)

    aw = jnp.stack([p['a_w_user'], p['a_w_item']])
    ab = jnp.stack([p['a_b_user'], p['a_b_item']])[:, None, :]
    sk = jnp.stack([p['skip_user'], p['skip_item']])[:, None, :]
    ls = jnp.stack([p['ln_s_user'], p['ln_s_item']])[:, None, :]
    lb = jnp.stack([p['ln_b_user'], p['ln_b_item']])[:, None, :]

    new_cat = _run_update(edge_msg, s_cat, h_cat, aw, ab, sk, ls, lb)
    return new_cat[:_N], new_cat[_N:]
